# baseline (speedup 1.0000x reference)
import functools

import jax
import jax.numpy as jnp
from jax import lax
from jax.experimental import pallas as pl
from jax.experimental.pallas import tpu as pltpu
from jax.experimental.pallas import tpu_sc as plsc

_N = 10000
_E = 320000
_D = 128
_H = 64
_C = 40
_G = 64

_NC = 2
_NS = 16
_NW = _NC * _NS
_EW = _E // _NW
_K = 80
_CH = _EW // _K

_NP = 10112
_RPT = _NP // _NS

_R = _NP
_DW = 16


def _sc_mesh():
    return plsc.VectorSubcoreMesh(
        core_axis_name="c", subcore_axis_name="s",
        num_cores=_NC, num_subcores=_NS)


def _deg_body(dst_hbm, zeros_hbm, ones_hbm, out_hbm, dst_v, ones_v, acc_sh):
    c = lax.axis_index("c")
    s = lax.axis_index("s")
    w = c * _NS + s
    pltpu.sync_copy(zeros_hbm, acc_sh.at[pl.ds(s * _RPT, _RPT)])
    pltpu.sync_copy(ones_hbm, ones_v)
    pltpu.sync_copy(dst_hbm.at[w], dst_v)
    plsc.subcore_barrier()

    @pl.loop(0, _CH)
    def _(i):
        pltpu.sync_copy(ones_v, acc_sh.at[dst_v.at[i]], add=True)

    plsc.subcore_barrier()
    pltpu.sync_copy(acc_sh.at[pl.ds(s * _RPT, _RPT)],
                    out_hbm.at[c, pl.ds(s * _RPT, _RPT)])


def _degrees(dst3, zeros_deg, ones_k):
    f = pl.kernel(
        _deg_body,
        out_type=jax.ShapeDtypeStruct((_NC, _NP, _DW), jnp.float32),
        mesh=_sc_mesh(),
        compiler_params=pltpu.CompilerParams(use_tc_tiling_on_sc=False),
        scratch_types=[
            pltpu.VMEM((_CH, _K), jnp.int32),
            pltpu.VMEM((_K, _DW), jnp.float32),
            pltpu.VMEM_SHARED((_NP, _DW), jnp.float32),
        ],
    )
    return f(dst3, zeros_deg, ones_k)


_NBUF = 5


def _agg_body(g_hbm, src_hbm, dst_hbm, zeros_hbm, out_hbm,
              src_v, dst_v, rows, acc_sh, gsems, ssems):
    c = lax.axis_index("c")
    s = lax.axis_index("s")
    w = c * _NS + s
    pltpu.sync_copy(zeros_hbm, acc_sh.at[pl.ds(s * _RPT, _RPT)])
    pltpu.sync_copy(src_hbm.at[w], src_v)
    pltpu.sync_copy(dst_hbm.at[w], dst_v)
    plsc.subcore_barrier()

    def start_g(i, b):
        pltpu.async_copy(g_hbm.at[src_v.at[i]], rows[b], gsems[b])

    def wait_g(i, b):
        pltpu.make_async_copy(g_hbm.at[src_v.at[i]], rows[b], gsems[b]).wait()

    def start_s(i, b):
        pltpu.async_copy(rows[b], acc_sh.at[dst_v.at[i]], ssems[b], add=True)

    def wait_s(i, b):
        pltpu.make_async_copy(rows[b], acc_sh.at[dst_v.at[i]],
                              ssems[b]).wait()

    for b in range(_NBUF):
        start_g(b, b)

    @pl.loop(0, _CH // _NBUF)
    def _(p):
        i0 = p * _NBUF
        for b in range(_NBUF):
            i = i0 + b
            wait_g(i, b)
            start_s(i, b)
            b_prev = (b - 1) % _NBUF
            i_prev = i - 1

            @pl.when((i_prev >= 0) & (i_prev + _NBUF < _CH))
            def _():
                wait_s(i_prev, b_prev)
                start_g(i_prev + _NBUF, b_prev)

    for b in range(_NBUF):
        wait_s(_CH - _NBUF + b, b)

    plsc.subcore_barrier()
    pltpu.sync_copy(acc_sh.at[pl.ds(s * _RPT, _RPT)],
                    out_hbm.at[c, pl.ds(s * _RPT, _RPT)])


def _aggregate(g, src3, dst3, zeros_rows):
    f = pl.kernel(
        _agg_body,
        out_type=jax.ShapeDtypeStruct((_NC, _NP, _H), jnp.float32),
        mesh=_sc_mesh(),
        compiler_params=pltpu.CompilerParams(use_tc_tiling_on_sc=False),
        scratch_types=[
            pltpu.VMEM((_CH, _K), jnp.int32),
            pltpu.VMEM((_CH, _K), jnp.int32),
            [pltpu.VMEM((_K, _H), jnp.float32) for _ in range(_NBUF)],
            pltpu.VMEM_SHARED((_NP, _H), jnp.float32),
            [pltpu.SemaphoreType.DMA for _ in range(_NBUF)],
            [pltpu.SemaphoreType.DMA for _ in range(_NBUF)],
        ],
    )
    return f(g, src3, dst3, zeros_rows)


def _first_body(degp, x, W1, dinv_out, g1_out):
    di = lax.rsqrt(degp[0, :, 0:1] + degp[1, :, 0:1] + 1.0)
    h = jnp.dot(x[...], W1[...], preferred_element_type=jnp.float32)
    dinv_out[...] = di
    g1_out[...] = di * h


def _first_layer(degp, x, W1):
    return pl.pallas_call(
        _first_body,
        grid=(_NP // _R,),
        in_specs=[
            pl.BlockSpec((_NC, _R, _DW), lambda i: (0, i, 0)),
            pl.BlockSpec((_R, _D), lambda i: (i, 0)),
            pl.BlockSpec((_D, _H), lambda i: (0, 0)),
        ],
        out_specs=[
            pl.BlockSpec((_R, 1), lambda i: (i, 0)),
            pl.BlockSpec((_R, _H), lambda i: (i, 0)),
        ],
        out_shape=[
            jax.ShapeDtypeStruct((_NP, 1), jnp.float32),
            jax.ShapeDtypeStruct((_NP, _H), jnp.float32),
        ],
    )(degp, x, W1)


def _mid_body(acc, g, dinv, b, W, gn_out):
    a = acc[0] + acc[1]
    y = jnp.maximum(dinv[...] * (a + g[...]) + b[...], 0.0)
    gn_out[...] = dinv[...] * jnp.dot(y, W[...],
                                      preferred_element_type=jnp.float32)


def _mid_layer(acc, g, dinv, b, W):
    return pl.pallas_call(
        _mid_body,
        grid=(_NP // _R,),
        in_specs=[
            pl.BlockSpec((_NC, _R, _H), lambda i: (0, i, 0)),
            pl.BlockSpec((_R, _H), lambda i: (i, 0)),
            pl.BlockSpec((_R, 1), lambda i: (i, 0)),
            pl.BlockSpec((1, _H), lambda i: (0, 0)),
            pl.BlockSpec((_H, _H), lambda i: (0, 0)),
        ],
        out_specs=pl.BlockSpec((_R, _H), lambda i: (i, 0)),
        out_shape=jax.ShapeDtypeStruct((_NP, _H), jnp.float32),
    )(acc, g, dinv, b, W)


def _pool_body(acc, g, dinv, b, batch, Wl, bl, out, sums, counts):
    i = pl.program_id(0)

    @pl.when(i == 0)
    def _():
        sums[...] = jnp.zeros_like(sums)
        counts[...] = jnp.zeros_like(counts)

    y = jnp.maximum(dinv[...] * (acc[0] + acc[1] + g[...]) + b[...], 0.0)
    oh = (lax.broadcasted_iota(jnp.int32, (_R, _G), 1)
          == batch[...]).astype(jnp.float32)
    cdims = (((0,), (0,)), ((), ()))
    sums[...] += lax.dot_general(oh, y, cdims,
                                 preferred_element_type=jnp.float32)
    counts[...] += lax.dot_general(oh, jnp.ones((_R, 1), jnp.float32), cdims,
                                   preferred_element_type=jnp.float32)

    @pl.when(i == pl.num_programs(0) - 1)
    def _():
        pooled = sums[...] / jnp.maximum(counts[...], 1.0)
        out[...] = jnp.dot(pooled, Wl[...],
                           preferred_element_type=jnp.float32) + bl[...]


def _pool_classify(acc, g, dinv, b, batch2d, Wl, bl):
    return pl.pallas_call(
        _pool_body,
        grid=(_NP // _R,),
        in_specs=[
            pl.BlockSpec((_NC, _R, _H), lambda i: (0, i, 0)),
            pl.BlockSpec((_R, _H), lambda i: (i, 0)),
            pl.BlockSpec((_R, 1), lambda i: (i, 0)),
            pl.BlockSpec((1, _H), lambda i: (0, 0)),
            pl.BlockSpec((_R, 1), lambda i: (i, 0)),
            pl.BlockSpec((_H, _C), lambda i: (0, 0)),
            pl.BlockSpec((1, _C), lambda i: (0, 0)),
        ],
        out_specs=pl.BlockSpec((_G, _C), lambda i: (0, 0)),
        out_shape=jax.ShapeDtypeStruct((_G, _C), jnp.float32),
        scratch_shapes=[
            pltpu.VMEM((_G, _H), jnp.float32),
            pltpu.VMEM((_G, 1), jnp.float32),
        ],
    )(acc, g, dinv, b, batch2d, Wl, bl)


def kernel(x, edge_index, batch, W1, b1, W2, b2, W3, b3, Wl, bl):
    src3 = edge_index[0].reshape(_NW, _CH, _K)
    dst3 = edge_index[1].reshape(_NW, _CH, _K)
    x = jnp.pad(x, ((0, _NP - _N), (0, 0)))
    batch = jnp.pad(batch, (0, _NP - _N), constant_values=_G)
    zeros_deg = jnp.zeros((_RPT, _DW), jnp.float32)
    ones_k = jnp.ones((_K, _DW), jnp.float32)
    zeros_rows = jnp.zeros((_RPT, _H), jnp.float32)

    degp = _degrees(dst3, zeros_deg, ones_k)
    dinv, g1 = _first_layer(degp, x, W1)
    acc1 = _aggregate(g1, src3, dst3, zeros_rows)
    g2 = _mid_layer(acc1, g1, dinv, b1.reshape(1, _H), W2)
    acc2 = _aggregate(g2, src3, dst3, zeros_rows)
    g3 = _mid_layer(acc2, g2, dinv, b2.reshape(1, _H), W3)
    acc3 = _aggregate(g3, src3, dst3, zeros_rows)
    out = _pool_classify(acc3, g3, dinv, b3.reshape(1, _H),
                         batch.reshape(_NP, 1), Wl, bl.reshape(1, _C))
    return out


# speedup vs baseline: 1.0385x; 1.0385x over previous
import functools

import jax
import jax.numpy as jnp
from jax import lax
from jax.experimental import pallas as pl
from jax.experimental.pallas import tpu as pltpu
from jax.experimental.pallas import tpu_sc as plsc

_N = 10000
_E = 320000
_D = 128
_H = 64
_C = 40
_G = 64

_NC = 2
_NS = 16
_NW = _NC * _NS
_EW = _E // _NW
_K = 80
_CH = _EW // _K

_NP = 10112
_RPT = _NP // _NS

_R = _NP
_DW = 16


def _sc_mesh():
    return plsc.VectorSubcoreMesh(
        core_axis_name="c", subcore_axis_name="s",
        num_cores=_NC, num_subcores=_NS)


def _deg_body(dst_hbm, zeros_hbm, ones_hbm, out_hbm, dst_v, ones_v, acc_sh,
              sem):
    c = lax.axis_index("c")
    s = lax.axis_index("s")
    w = c * _NS + s
    pltpu.sync_copy(zeros_hbm, acc_sh.at[pl.ds(s * _RPT, _RPT)])
    pltpu.sync_copy(ones_hbm, ones_v)
    pltpu.sync_copy(dst_hbm.at[w], dst_v)
    plsc.subcore_barrier()

    @pl.loop(0, _CH)
    def _(i):
        pltpu.async_copy(ones_v, acc_sh.at[dst_v.at[i]], sem, add=True)

    @pl.loop(0, _CH)
    def _(i):
        pltpu.make_async_copy(ones_v, acc_sh.at[dst_v.at[i]], sem).wait()

    plsc.subcore_barrier()
    pltpu.sync_copy(acc_sh.at[pl.ds(s * _RPT, _RPT)],
                    out_hbm.at[c, pl.ds(s * _RPT, _RPT)])


def _degrees(dst3, zeros_deg, ones_k):
    f = pl.kernel(
        _deg_body,
        out_type=jax.ShapeDtypeStruct((_NC, _NP, _DW), jnp.float32),
        mesh=_sc_mesh(),
        compiler_params=pltpu.CompilerParams(use_tc_tiling_on_sc=False),
        scratch_types=[
            pltpu.VMEM((_CH, _K), jnp.int32),
            pltpu.VMEM((_K, _DW), jnp.float32),
            pltpu.VMEM_SHARED((_NP, _DW), jnp.float32),
            pltpu.SemaphoreType.DMA,
        ],
    )
    return f(dst3, zeros_deg, ones_k)


_NBUF = 5


def _agg_body(g_hbm, src_hbm, dst_hbm, zeros_hbm, out_hbm,
              src_v, dst_v, rows, acc_sh, gsems):
    c = lax.axis_index("c")
    s = lax.axis_index("s")
    w = c * _NS + s
    pltpu.sync_copy(zeros_hbm, acc_sh.at[pl.ds(s * _RPT, _RPT)])
    pltpu.sync_copy(src_hbm.at[w], src_v)
    pltpu.sync_copy(dst_hbm.at[w], dst_v)
    plsc.subcore_barrier()

    def start(i, b):
        pltpu.async_copy(g_hbm.at[src_v.at[i]], rows[b], gsems[b])

    def finish(i, b):
        pltpu.make_async_copy(g_hbm.at[src_v.at[i]], rows[b], gsems[b]).wait()
        pltpu.sync_copy(rows[b], acc_sh.at[dst_v.at[i]], add=True)

    for b in range(_NBUF):
        start(b, b)

    @pl.loop(0, _CH // _NBUF)
    def _(p):
        i0 = p * _NBUF
        for b in range(_NBUF):
            i = i0 + b
            finish(i, b)

            @pl.when(i + _NBUF < _CH)
            def _():
                start(i + _NBUF, b)

    plsc.subcore_barrier()
    pltpu.sync_copy(acc_sh.at[pl.ds(s * _RPT, _RPT)],
                    out_hbm.at[c, pl.ds(s * _RPT, _RPT)])


def _aggregate(g, src3, dst3, zeros_rows):
    f = pl.kernel(
        _agg_body,
        out_type=jax.ShapeDtypeStruct((_NC, _NP, _H), jnp.float32),
        mesh=_sc_mesh(),
        compiler_params=pltpu.CompilerParams(use_tc_tiling_on_sc=False),
        scratch_types=[
            pltpu.VMEM((_CH, _K), jnp.int32),
            pltpu.VMEM((_CH, _K), jnp.int32),
            [pltpu.VMEM((_K, _H), jnp.float32) for _ in range(_NBUF)],
            pltpu.VMEM_SHARED((_NP, _H), jnp.float32),
            [pltpu.SemaphoreType.DMA for _ in range(_NBUF)],
        ],
    )
    return f(g, src3, dst3, zeros_rows)


def _first_body(degp, x, W1, dinv_out, g1_out):
    di = lax.rsqrt(degp[0, :, 0:1] + degp[1, :, 0:1] + 1.0)
    h = jnp.dot(x[...], W1[...], preferred_element_type=jnp.float32)
    dinv_out[...] = di
    g1_out[...] = di * h


def _first_layer(degp, x, W1):
    return pl.pallas_call(
        _first_body,
        grid=(_NP // _R,),
        in_specs=[
            pl.BlockSpec((_NC, _R, _DW), lambda i: (0, i, 0)),
            pl.BlockSpec((_R, _D), lambda i: (i, 0)),
            pl.BlockSpec((_D, _H), lambda i: (0, 0)),
        ],
        out_specs=[
            pl.BlockSpec((_R, 1), lambda i: (i, 0)),
            pl.BlockSpec((_R, _H), lambda i: (i, 0)),
        ],
        out_shape=[
            jax.ShapeDtypeStruct((_NP, 1), jnp.float32),
            jax.ShapeDtypeStruct((_NP, _H), jnp.float32),
        ],
    )(degp, x, W1)


def _mid_body(acc, g, dinv, b, W, gn_out):
    a = acc[0] + acc[1]
    y = jnp.maximum(dinv[...] * (a + g[...]) + b[...], 0.0)
    gn_out[...] = dinv[...] * jnp.dot(y, W[...],
                                      preferred_element_type=jnp.float32)


def _mid_layer(acc, g, dinv, b, W):
    return pl.pallas_call(
        _mid_body,
        grid=(_NP // _R,),
        in_specs=[
            pl.BlockSpec((_NC, _R, _H), lambda i: (0, i, 0)),
            pl.BlockSpec((_R, _H), lambda i: (i, 0)),
            pl.BlockSpec((_R, 1), lambda i: (i, 0)),
            pl.BlockSpec((1, _H), lambda i: (0, 0)),
            pl.BlockSpec((_H, _H), lambda i: (0, 0)),
        ],
        out_specs=pl.BlockSpec((_R, _H), lambda i: (i, 0)),
        out_shape=jax.ShapeDtypeStruct((_NP, _H), jnp.float32),
    )(acc, g, dinv, b, W)


def _pool_body(acc, g, dinv, b, batch, Wl, bl, out, sums, counts):
    i = pl.program_id(0)

    @pl.when(i == 0)
    def _():
        sums[...] = jnp.zeros_like(sums)
        counts[...] = jnp.zeros_like(counts)

    y = jnp.maximum(dinv[...] * (acc[0] + acc[1] + g[...]) + b[...], 0.0)
    oh = (lax.broadcasted_iota(jnp.int32, (_R, _G), 1)
          == batch[...]).astype(jnp.float32)
    cdims = (((0,), (0,)), ((), ()))
    sums[...] += lax.dot_general(oh, y, cdims,
                                 preferred_element_type=jnp.float32)
    counts[...] += lax.dot_general(oh, jnp.ones((_R, 1), jnp.float32), cdims,
                                   preferred_element_type=jnp.float32)

    @pl.when(i == pl.num_programs(0) - 1)
    def _():
        pooled = sums[...] / jnp.maximum(counts[...], 1.0)
        out[...] = jnp.dot(pooled, Wl[...],
                           preferred_element_type=jnp.float32) + bl[...]


def _pool_classify(acc, g, dinv, b, batch2d, Wl, bl):
    return pl.pallas_call(
        _pool_body,
        grid=(_NP // _R,),
        in_specs=[
            pl.BlockSpec((_NC, _R, _H), lambda i: (0, i, 0)),
            pl.BlockSpec((_R, _H), lambda i: (i, 0)),
            pl.BlockSpec((_R, 1), lambda i: (i, 0)),
            pl.BlockSpec((1, _H), lambda i: (0, 0)),
            pl.BlockSpec((_R, 1), lambda i: (i, 0)),
            pl.BlockSpec((_H, _C), lambda i: (0, 0)),
            pl.BlockSpec((1, _C), lambda i: (0, 0)),
        ],
        out_specs=pl.BlockSpec((_G, _C), lambda i: (0, 0)),
        out_shape=jax.ShapeDtypeStruct((_G, _C), jnp.float32),
        scratch_shapes=[
            pltpu.VMEM((_G, _H), jnp.float32),
            pltpu.VMEM((_G, 1), jnp.float32),
        ],
    )(acc, g, dinv, b, batch2d, Wl, bl)


def kernel(x, edge_index, batch, W1, b1, W2, b2, W3, b3, Wl, bl):
    src3 = edge_index[0].reshape(_NW, _CH, _K)
    dst3 = edge_index[1].reshape(_NW, _CH, _K)
    x = jnp.pad(x, ((0, _NP - _N), (0, 0)))
    batch = jnp.pad(batch, (0, _NP - _N), constant_values=_G)
    zeros_deg = jnp.zeros((_RPT, _DW), jnp.float32)
    ones_k = jnp.ones((_K, _DW), jnp.float32)
    zeros_rows = jnp.zeros((_RPT, _H), jnp.float32)

    degp = _degrees(dst3, zeros_deg, ones_k)
    dinv, g1 = _first_layer(degp, x, W1)
    acc1 = _aggregate(g1, src3, dst3, zeros_rows)
    g2 = _mid_layer(acc1, g1, dinv, b1.reshape(1, _H), W2)
    acc2 = _aggregate(g2, src3, dst3, zeros_rows)
    g3 = _mid_layer(acc2, g2, dinv, b2.reshape(1, _H), W3)
    acc3 = _aggregate(g3, src3, dst3, zeros_rows)
    out = _pool_classify(acc3, g3, dinv, b3.reshape(1, _H),
                         batch.reshape(_NP, 1), Wl, bl.reshape(1, _C))
    return out
